# Initial kernel scaffold; baseline (speedup 1.0000x reference)
import jax
import jax.numpy as jnp
from jax.experimental import pallas as pl


def kernel(nodes, neigh_indices, num_sample, features):
    raise NotImplementedError("write your pallas kernel here")


# speedup vs baseline: 1.9376x; 1.9376x over previous
import math

import jax
import jax.numpy as jnp
from jax import lax
from jax.experimental import pallas as pl
from jax.experimental.pallas import tpu as pltpu
from jax.experimental.pallas import tpu_sc as plsc

NC = 2
NS = 16
NW = NC * NS
LANES = 16


def _build_sc_call(B_pad, S, D, n_chunks, chunk_rows, scale):
    idx_rows_per_chunk = (chunk_rows * S) // 128
    rows_per_worker = n_chunks * chunk_rows
    idx_rows_per_worker = n_chunks * idx_rows_per_chunk
    mesh = plsc.VectorSubcoreMesh(
        core_axis_name="c", subcore_axis_name="s", num_cores=NC, num_subcores=NS
    )

    i32 = jnp.int32

    def body(feat_hbm, idx_hbm, out_hbm, idx_v, rows_v, out_v, sem):
        wid = lax.axis_index("s") * i32(NC) + lax.axis_index("c")
        pltpu.sync_copy(idx_hbm.at[wid], idx_v)

        def chunk_body(ci, carry):
            row0 = wid * i32(rows_per_worker) + ci * i32(chunk_rows)
            copies = [
                pltpu.async_copy(
                    feat_hbm.at[idx_v.at[ci * i32(idx_rows_per_chunk) + i32(g)]],
                    rows_v.at[pl.ds(g * 128, 128)],
                    sem,
                )
                for g in range(idx_rows_per_chunk)
            ]
            for cp in copies:
                cp.wait()

            def row_body(r, inner_carry):
                base = r * i32(S)
                for d in range(D // LANES):
                    sl = pl.ds(d * LANES, LANES)
                    acc = rows_v[base, sl]
                    for j in range(1, S):
                        acc = acc + rows_v[base + i32(j), sl]
                    out_v[r, sl] = acc * scale
                return inner_carry

            lax.fori_loop(i32(0), i32(chunk_rows), row_body, i32(0))
            pltpu.sync_copy(out_v, out_hbm.at[pl.ds(row0, chunk_rows)])
            return carry

        lax.fori_loop(i32(0), i32(n_chunks), chunk_body, i32(0))

    return pl.kernel(
        body,
        out_type=jax.ShapeDtypeStruct((B_pad, D), jnp.float32),
        mesh=mesh,
        scratch_types=[
            pltpu.VMEM((idx_rows_per_worker, 128), jnp.int32),
            pltpu.VMEM((chunk_rows * S, D), jnp.float32),
            pltpu.VMEM((chunk_rows, D), jnp.float32),
            pltpu.SemaphoreType.DMA,
        ],
    )


def kernel(nodes, neigh_indices, num_sample, features):
    del nodes
    B, S = neigh_indices.shape
    N, D = features.shape
    assert D % LANES == 0

    chunk_rows = 128 // math.gcd(S, 128)
    block = NW * chunk_rows
    B_pad = block * ((B + block - 1) // block)
    n_chunks = B_pad // block

    flat_idx = neigh_indices.astype(jnp.int32).reshape(-1)
    pad = B_pad * S - flat_idx.shape[0]
    if pad:
        flat_idx = jnp.concatenate([flat_idx, jnp.zeros((pad,), jnp.int32)])
    idx3d = flat_idx.reshape(NW, B_pad * S // NW // 128, 128)

    feats = features.astype(jnp.float32)
    scale = jnp.float32(1.0 / num_sample)

    call = _build_sc_call(B_pad, S, D, n_chunks, chunk_rows, scale)
    out = call(feats, idx3d)
    return out[:B]
